# Initial kernel scaffold; baseline (speedup 1.0000x reference)
import jax
import jax.numpy as jnp
from jax.experimental import pallas as pl


def kernel(token_ids, embeddings):
    raise NotImplementedError("write your pallas kernel here")


# speedup vs baseline: 1.5013x; 1.5013x over previous
import functools

import jax
import jax.numpy as jnp
from jax import lax
from jax.experimental import pallas as pl
from jax.experimental.pallas import tpu as pltpu
from jax.experimental.pallas import tpu_sc as plsc

_D = 32
_CHUNK = 128
_G = 8
_ROWS = _G * _CHUNK


def _make_lookup(total, num_workers, num_cores):
    bpw = total // num_workers
    nch = bpw // _CHUNK
    nsup = nch // _G
    mesh = plsc.VectorSubcoreMesh(core_axis_name="c", subcore_axis_name="s")

    @functools.partial(
        pl.kernel,
        out_type=jax.ShapeDtypeStruct((total, _D), jnp.float32),
        mesh=mesh,
        scratch_types=[
            pltpu.VMEM((nch, _CHUNK), jnp.int32),
            pltpu.VMEM((_ROWS, _D), jnp.float32),
            pltpu.SemaphoreType.DMA,
        ],
        compiler_params=pltpu.CompilerParams(use_tc_tiling_on_sc=False),
    )
    def lookup(ids_hbm, table_hbm, out_hbm, idx_v, rows_v, sem):
        wid = lax.axis_index("s") * num_cores + lax.axis_index("c")
        base = wid * bpw
        pltpu.sync_copy(ids_hbm.at[wid], idx_v)

        def body(s, _):
            handles = []
            for g in range(_G):
                j = s * _G + g
                handles.append(
                    pltpu.async_copy(
                        table_hbm.at[idx_v.at[j]],
                        rows_v.at[pl.ds(g * _CHUNK, _CHUNK)],
                        sem,
                    )
                )
            for h in handles:
                h.wait()
            pltpu.sync_copy(rows_v, out_hbm.at[pl.ds(base + s * _ROWS, _ROWS)])
            return ()

        lax.fori_loop(0, nsup, body, (), unroll=False)

    return lookup


def kernel(token_ids, embeddings):
    b, t = token_ids.shape
    total = b * t
    info = plsc.get_sparse_core_info()
    nw = info.num_cores * info.num_subcores
    ids = token_ids.reshape(nw, total // (nw * _CHUNK), _CHUNK)
    out = _make_lookup(total, nw, info.num_cores)(ids, embeddings)
    return out.reshape(b, t, _D)


# speedup vs baseline: 1.5097x; 1.0056x over previous
import functools

import jax
import jax.numpy as jnp
from jax import lax
from jax.experimental import pallas as pl
from jax.experimental.pallas import tpu as pltpu
from jax.experimental.pallas import tpu_sc as plsc

_D = 32
_CHUNK = 1024
_G = 2
_ROWS = _G * _CHUNK


def _make_lookup(total, num_workers, num_cores):
    bpw = total // num_workers
    nch = bpw // _CHUNK
    nsup = nch // _G
    mesh = plsc.VectorSubcoreMesh(core_axis_name="c", subcore_axis_name="s")

    @functools.partial(
        pl.kernel,
        out_type=jax.ShapeDtypeStruct((total, _D), jnp.float32),
        mesh=mesh,
        scratch_types=[
            pltpu.VMEM((nch, _CHUNK), jnp.int32),
            pltpu.VMEM((_ROWS, _D), jnp.float32),
            pltpu.SemaphoreType.DMA,
        ],
        compiler_params=pltpu.CompilerParams(use_tc_tiling_on_sc=False),
    )
    def lookup(ids_hbm, table_hbm, out_hbm, idx_v, rows_v, sem):
        wid = lax.axis_index("s") * num_cores + lax.axis_index("c")
        base = wid * bpw
        pltpu.sync_copy(ids_hbm.at[wid], idx_v)

        def body(s, _):
            handles = []
            for g in range(_G):
                j = s * _G + g
                handles.append(
                    pltpu.async_copy(
                        table_hbm.at[idx_v.at[j]],
                        rows_v.at[pl.ds(g * _CHUNK, _CHUNK)],
                        sem,
                    )
                )
            for h in handles:
                h.wait()
            pltpu.sync_copy(rows_v, out_hbm.at[pl.ds(base + s * _ROWS, _ROWS)])
            return ()

        lax.fori_loop(0, nsup, body, (), unroll=False)

    return lookup


def kernel(token_ids, embeddings):
    b, t = token_ids.shape
    total = b * t
    info = plsc.get_sparse_core_info()
    nw = info.num_cores * info.num_subcores
    ids = token_ids.reshape(nw, total // (nw * _CHUNK), _CHUNK)
    out = _make_lookup(total, nw, info.num_cores)(ids, embeddings)
    return out.reshape(b, t, _D)


# speedup vs baseline: 1.5119x; 1.0015x over previous
import functools

import jax
import jax.numpy as jnp
from jax import lax
from jax.experimental import pallas as pl
from jax.experimental.pallas import tpu as pltpu
from jax.experimental.pallas import tpu_sc as plsc

_D = 32
_CHUNK = 1280


def _make_lookup(total, num_workers, num_cores):
    bpw = total // num_workers
    nsup = bpw // _CHUNK
    mesh = plsc.VectorSubcoreMesh(core_axis_name="c", subcore_axis_name="s")

    @functools.partial(
        pl.kernel,
        out_type=jax.ShapeDtypeStruct((total, _D), jnp.float32),
        mesh=mesh,
        scratch_types=[
            pltpu.VMEM((nsup, _CHUNK), jnp.int32),
            pltpu.VMEM((2, _CHUNK, _D), jnp.float32),
            pltpu.SemaphoreType.DMA,
            pltpu.SemaphoreType.DMA,
            pltpu.SemaphoreType.DMA,
        ],
        compiler_params=pltpu.CompilerParams(use_tc_tiling_on_sc=False),
    )
    def lookup(ids_hbm, table_hbm, out_hbm, idx_v, rows_v, gsem0, gsem1, osem):
        wid = lax.axis_index("s") * num_cores + lax.axis_index("c")
        base = wid * bpw
        gsems = (gsem0, gsem1)
        pltpu.sync_copy(ids_hbm.at[wid], idx_v)

        def start_gather(s):
            return pltpu.async_copy(
                table_hbm.at[idx_v.at[s]], rows_v.at[s % 2], gsems[s % 2]
            )

        h_g = [None] * nsup
        h_o = [None] * nsup
        h_g[0] = start_gather(0)
        for s in range(nsup):
            if s + 1 < nsup:
                if s >= 1:
                    h_o[s - 1].wait()
                h_g[s + 1] = start_gather(s + 1)
            h_g[s].wait()
            h_o[s] = pltpu.async_copy(
                rows_v.at[s % 2], out_hbm.at[pl.ds(base + s * _CHUNK, _CHUNK)], osem
            )
        h_o[nsup - 1].wait()

    return lookup


def kernel(token_ids, embeddings):
    b, t = token_ids.shape
    total = b * t
    info = plsc.get_sparse_core_info()
    nw = info.num_cores * info.num_subcores
    ids = token_ids.reshape(nw, total // (nw * _CHUNK), _CHUNK)
    out = _make_lookup(total, nw, info.num_cores)(ids, embeddings)
    return out.reshape(b, t, _D)
